# baseline (speedup 1.0000x reference)
import functools
import math

import jax
import jax.numpy as jnp
from jax import lax
from jax.experimental import pallas as pl
from jax.experimental.pallas import tpu as pltpu
from jax.experimental.pallas import tpu_sc as plsc

_LANES = 16
_IDX_W = 128


def _embed_sc(tokens_2d, table, scale):
    n_rows, idx_w = tokens_2d.shape
    vocab, dim = table.shape
    info = plsc.get_sparse_core_info()
    n_workers = info.num_cores * info.num_subcores
    rows_per_w = n_rows // n_workers
    b_per_w = rows_per_w * idx_w
    total = n_rows * idx_w

    mesh = plsc.VectorSubcoreMesh(core_axis_name="c", subcore_axis_name="s")

    @functools.partial(
        pl.kernel,
        mesh=mesh,
        out_type=jax.ShapeDtypeStruct((total, dim), jnp.float32),
        scratch_types=[
            pltpu.VMEM((rows_per_w, idx_w), jnp.int32),
            pltpu.VMEM((_IDX_W, dim), jnp.float32),
            pltpu.SemaphoreType.DMA,
        ],
        compiler_params=pltpu.CompilerParams(use_tc_tiling_on_sc=False),
    )
    def k(tok_hbm, tab_hbm, out_hbm, idx_v, rows_v, sem):
        wid = lax.axis_index("s") * info.num_cores + lax.axis_index("c")
        base = wid * b_per_w
        pltpu.sync_copy(tok_hbm.at[pl.ds(wid * rows_per_w, rows_per_w)], idx_v)

        def chunk(j, _):
            pltpu.async_copy(tab_hbm.at[idx_v.at[j]], rows_v, sem).wait()

            def scale_row(i, _):
                for d0 in range(dim // _LANES):
                    sl = pl.ds(d0 * _LANES, _LANES)
                    rows_v[i, sl] = rows_v[i, sl] * scale
                return 0

            lax.fori_loop(0, _IDX_W, scale_row, 0)
            pltpu.sync_copy(rows_v, out_hbm.at[pl.ds(base + j * _IDX_W, _IDX_W)])
            return 0

        lax.fori_loop(0, rows_per_w, chunk, 0)

    return k(tokens_2d, table)


def kernel(tokens, embedding_weight):
    b0, b1 = tokens.shape
    vocab, dim = embedding_weight.shape
    scale = math.sqrt(dim)
    total = b0 * b1
    toks = tokens.reshape(total // _IDX_W, _IDX_W)
    out = _embed_sc(toks, embedding_weight, scale)
    return out.reshape(b0, b1, dim)
